# Initial kernel scaffold; baseline (speedup 1.0000x reference)
import jax
import jax.numpy as jnp
from jax.experimental import pallas as pl


def kernel(w, b, x):
    raise NotImplementedError("write your pallas kernel here")


# speedup vs baseline: 1.3339x; 1.3339x over previous
import functools

import jax
import jax.numpy as jnp
from jax import lax
from jax.experimental import pallas as pl
from jax.experimental.pallas import tpu as pltpu


def _ph_kernel(C, H, W, sW, x_ref, w_ref, b_ref, o_ref):
    acc = x_ref[0, 0].astype(jnp.float32) * w_ref[0]
    for c in range(1, C):
        acc = acc + x_ref[0, c].astype(jnp.float32) * w_ref[c]

    r_w = (W - 1) / (sW - 1)
    win = lax.broadcasted_iota(jnp.int32, (W, sW), 0).astype(jnp.float32)
    wout = lax.broadcasted_iota(jnp.int32, (W, sW), 1).astype(jnp.float32)
    src_w = jnp.minimum(wout * r_w, W - 1)
    mwt = jnp.maximum(0.0, 1.0 - jnp.abs(src_w - win))
    mid = jnp.dot(acc, mwt, preferred_element_type=jnp.float32)

    r_h = (H - 1) / (2 * H - 1)
    bias = b_ref[0]
    p = mid[:-1]
    q = mid[1:]
    ii = (lax.broadcasted_iota(jnp.int32, (H - 1, 1), 0)
          .astype(jnp.float32) + 1.0)
    src_e = (2.0 * ii) * r_h
    wa = 1.0 - (src_e - (ii - 1.0))
    wb = 1.0 - (ii - src_e)
    e_body = p * wa + q * wb
    jj = (lax.broadcasted_iota(jnp.int32, (H - 1, 1), 0)
          .astype(jnp.float32))
    src_o = (2.0 * jj + 1.0) * r_h
    wc = 1.0 - (src_o - jj)
    wd = 1.0 - ((jj + 1.0) - src_o)
    o_body = p * wc + q * wd
    e_rows = jnp.concatenate([mid[0:1], e_body], axis=0) + bias
    o_rows = jnp.concatenate([o_body, mid[H - 1:H]], axis=0) + bias
    o_ref[0, 0, :, 0:sW] = e_rows.astype(o_ref.dtype)
    o_ref[0, 0, :, sW:2 * sW] = o_rows.astype(o_ref.dtype)


def kernel(w, b, x):
    B, C, H, W = x.shape
    sH, sW = 2 * H, 2 * W
    wv = w.astype(jnp.float32).reshape(C)
    bv = b.astype(jnp.float32).reshape(1)
    smem = pl.BlockSpec(memory_space=pltpu.MemorySpace.SMEM)
    out = pl.pallas_call(
        functools.partial(_ph_kernel, C, H, W, sW),
        out_shape=jax.ShapeDtypeStruct((B, 1, H, 2 * sW), jnp.float32),
        grid=(B,),
        in_specs=[pl.BlockSpec((1, C, H, W), lambda i: (i, 0, 0, 0)),
                  smem, smem],
        out_specs=pl.BlockSpec((1, 1, H, 2 * sW), lambda i: (i, 0, 0, 0)),
        compiler_params=pltpu.CompilerParams(
            dimension_semantics=("parallel",),
            vmem_limit_bytes=64 * 1024 * 1024),
    )(x, wv, bv)
    return out.reshape(B, 1, sH, sW)


# speedup vs baseline: 1.5592x; 1.1689x over previous
import functools

import jax
import jax.numpy as jnp
from jax import lax
from jax.experimental import pallas as pl
from jax.experimental.pallas import tpu as pltpu


_IMGS_PER_STEP = 2


def _ph_kernel(G, C, H, W, sW, x_ref, w_ref, b_ref, o_ref):
    r_w = (W - 1) / (sW - 1)
    win = lax.broadcasted_iota(jnp.int32, (W, sW), 0).astype(jnp.float32)
    wout = lax.broadcasted_iota(jnp.int32, (W, sW), 1).astype(jnp.float32)
    src_w = jnp.minimum(wout * r_w, W - 1)
    mwt = jnp.maximum(0.0, 1.0 - jnp.abs(src_w - win))

    r_h = (H - 1) / (2 * H - 1)
    bias = b_ref[0]
    ii = (lax.broadcasted_iota(jnp.int32, (H - 1, 1), 0)
          .astype(jnp.float32) + 1.0)
    src_e = (2.0 * ii) * r_h
    wa = 1.0 - (src_e - (ii - 1.0))
    wb = 1.0 - (ii - src_e)
    jj = (lax.broadcasted_iota(jnp.int32, (H - 1, 1), 0)
          .astype(jnp.float32))
    src_o = (2.0 * jj + 1.0) * r_h
    wc = 1.0 - (src_o - jj)
    wd = 1.0 - ((jj + 1.0) - src_o)

    for g in range(G):
        acc = x_ref[g, 0].astype(jnp.float32) * w_ref[0]
        for c in range(1, C):
            acc = acc + x_ref[g, c].astype(jnp.float32) * w_ref[c]

        mid = jnp.dot(acc, mwt, preferred_element_type=jnp.float32)

        p = mid[:-1]
        q = mid[1:]
        e_body = p * wa + q * wb
        o_body = p * wc + q * wd
        e_rows = jnp.concatenate([mid[0:1], e_body], axis=0) + bias
        o_rows = jnp.concatenate([o_body, mid[H - 1:H]], axis=0) + bias
        o_ref[g, 0, :, 0:sW] = e_rows.astype(o_ref.dtype)
        o_ref[g, 0, :, sW:2 * sW] = o_rows.astype(o_ref.dtype)


def kernel(w, b, x):
    B, C, H, W = x.shape
    sH, sW = 2 * H, 2 * W
    G = _IMGS_PER_STEP if B % _IMGS_PER_STEP == 0 else 1
    wv = w.astype(jnp.float32).reshape(C)
    bv = b.astype(jnp.float32).reshape(1)
    smem = pl.BlockSpec(memory_space=pltpu.MemorySpace.SMEM)
    out = pl.pallas_call(
        functools.partial(_ph_kernel, G, C, H, W, sW),
        out_shape=jax.ShapeDtypeStruct((B, 1, H, 2 * sW), jnp.float32),
        grid=(B // G,),
        in_specs=[pl.BlockSpec((G, C, H, W), lambda i: (i, 0, 0, 0)),
                  smem, smem],
        out_specs=pl.BlockSpec((G, 1, H, 2 * sW), lambda i: (i, 0, 0, 0)),
        compiler_params=pltpu.CompilerParams(
            dimension_semantics=("parallel",),
            vmem_limit_bytes=96 * 1024 * 1024),
    )(x, wv, bv)
    return out.reshape(B, 1, sH, sW)


# speedup vs baseline: 1.6837x; 1.0798x over previous
import functools

import jax
import jax.numpy as jnp
from jax import lax
from jax.experimental import pallas as pl
from jax.experimental.pallas import tpu as pltpu


_IMGS_PER_STEP = 4


def _ph_kernel(G, C, H, W, sW, x_ref, w_ref, b_ref, o_ref):
    r_w = (W - 1) / (sW - 1)
    win = lax.broadcasted_iota(jnp.int32, (W, sW), 0).astype(jnp.float32)
    wout = lax.broadcasted_iota(jnp.int32, (W, sW), 1).astype(jnp.float32)
    src_w = jnp.minimum(wout * r_w, W - 1)
    mwt = jnp.maximum(0.0, 1.0 - jnp.abs(src_w - win))

    r_h = (H - 1) / (2 * H - 1)
    bias = b_ref[0]
    ii = (lax.broadcasted_iota(jnp.int32, (H - 1, 1), 0)
          .astype(jnp.float32) + 1.0)
    src_e = (2.0 * ii) * r_h
    wa = 1.0 - (src_e - (ii - 1.0))
    wb = 1.0 - (ii - src_e)
    jj = (lax.broadcasted_iota(jnp.int32, (H - 1, 1), 0)
          .astype(jnp.float32))
    src_o = (2.0 * jj + 1.0) * r_h
    wc = 1.0 - (src_o - jj)
    wd = 1.0 - ((jj + 1.0) - src_o)

    for g in range(G):
        acc = x_ref[g, 0].astype(jnp.float32) * w_ref[0]
        for c in range(1, C):
            acc = acc + x_ref[g, c].astype(jnp.float32) * w_ref[c]

        mid = jnp.dot(acc, mwt, preferred_element_type=jnp.float32)

        p = mid[:-1]
        q = mid[1:]
        e_body = p * wa + q * wb
        o_body = p * wc + q * wd
        e_rows = jnp.concatenate([mid[0:1], e_body], axis=0) + bias
        o_rows = jnp.concatenate([o_body, mid[H - 1:H]], axis=0) + bias
        o_ref[g, 0, :, 0:sW] = e_rows.astype(o_ref.dtype)
        o_ref[g, 0, :, sW:2 * sW] = o_rows.astype(o_ref.dtype)


def kernel(w, b, x):
    B, C, H, W = x.shape
    sH, sW = 2 * H, 2 * W
    G = _IMGS_PER_STEP if B % _IMGS_PER_STEP == 0 else 1
    wv = w.astype(jnp.float32).reshape(C)
    bv = b.astype(jnp.float32).reshape(1)
    smem = pl.BlockSpec(memory_space=pltpu.MemorySpace.SMEM)
    out = pl.pallas_call(
        functools.partial(_ph_kernel, G, C, H, W, sW),
        out_shape=jax.ShapeDtypeStruct((B, 1, H, 2 * sW), jnp.float32),
        grid=(B // G,),
        in_specs=[pl.BlockSpec((G, C, H, W), lambda i: (i, 0, 0, 0)),
                  smem, smem],
        out_specs=pl.BlockSpec((G, 1, H, 2 * sW), lambda i: (i, 0, 0, 0)),
        compiler_params=pltpu.CompilerParams(
            dimension_semantics=("parallel",),
            vmem_limit_bytes=96 * 1024 * 1024),
    )(x, wv, bv)
    return out.reshape(B, 1, sH, sW)


# speedup vs baseline: 1.7138x; 1.0179x over previous
import functools

import jax
import jax.numpy as jnp
from jax import lax
from jax.experimental import pallas as pl
from jax.experimental.pallas import tpu as pltpu


_IMGS_PER_STEP = 8


def _ph_kernel(G, C, H, W, sW, x_ref, w_ref, b_ref, o_ref):
    r_w = (W - 1) / (sW - 1)
    win = lax.broadcasted_iota(jnp.int32, (W, sW), 0).astype(jnp.float32)
    wout = lax.broadcasted_iota(jnp.int32, (W, sW), 1).astype(jnp.float32)
    src_w = jnp.minimum(wout * r_w, W - 1)
    mwt = jnp.maximum(0.0, 1.0 - jnp.abs(src_w - win))

    r_h = (H - 1) / (2 * H - 1)
    bias = b_ref[0]
    ii = (lax.broadcasted_iota(jnp.int32, (H - 1, 1), 0)
          .astype(jnp.float32) + 1.0)
    src_e = (2.0 * ii) * r_h
    wa = 1.0 - (src_e - (ii - 1.0))
    wb = 1.0 - (ii - src_e)
    jj = (lax.broadcasted_iota(jnp.int32, (H - 1, 1), 0)
          .astype(jnp.float32))
    src_o = (2.0 * jj + 1.0) * r_h
    wc = 1.0 - (src_o - jj)
    wd = 1.0 - ((jj + 1.0) - src_o)

    for g in range(G):
        acc = x_ref[g, 0].astype(jnp.float32) * w_ref[0]
        for c in range(1, C):
            acc = acc + x_ref[g, c].astype(jnp.float32) * w_ref[c]

        mid = jnp.dot(acc, mwt, preferred_element_type=jnp.float32)

        p = mid[:-1]
        q = mid[1:]
        e_body = p * wa + q * wb
        o_body = p * wc + q * wd
        e_rows = jnp.concatenate([mid[0:1], e_body], axis=0) + bias
        o_rows = jnp.concatenate([o_body, mid[H - 1:H]], axis=0) + bias
        o_ref[g, 0, :, 0:sW] = e_rows.astype(o_ref.dtype)
        o_ref[g, 0, :, sW:2 * sW] = o_rows.astype(o_ref.dtype)


def kernel(w, b, x):
    B, C, H, W = x.shape
    sH, sW = 2 * H, 2 * W
    G = _IMGS_PER_STEP if B % _IMGS_PER_STEP == 0 else 1
    wv = w.astype(jnp.float32).reshape(C)
    bv = b.astype(jnp.float32).reshape(1)
    smem = pl.BlockSpec(memory_space=pltpu.MemorySpace.SMEM)
    out = pl.pallas_call(
        functools.partial(_ph_kernel, G, C, H, W, sW),
        out_shape=jax.ShapeDtypeStruct((B, 1, H, 2 * sW), jnp.float32),
        grid=(B // G,),
        in_specs=[pl.BlockSpec((G, C, H, W), lambda i: (i, 0, 0, 0)),
                  smem, smem],
        out_specs=pl.BlockSpec((G, 1, H, 2 * sW), lambda i: (i, 0, 0, 0)),
        compiler_params=pltpu.CompilerParams(
            dimension_semantics=("parallel",),
            vmem_limit_bytes=96 * 1024 * 1024),
    )(x, wv, bv)
    return out.reshape(B, 1, sH, sW)


# speedup vs baseline: 1.7165x; 1.0016x over previous
import functools

import jax
import jax.numpy as jnp
from jax import lax
from jax.experimental import pallas as pl
from jax.experimental.pallas import tpu as pltpu


_IMGS_PER_STEP = 8


def _ph_kernel(G, C, H, W, sW, x_ref, w_ref, b_ref, o_ref):
    r_w = (W - 1) / (sW - 1)
    win = lax.broadcasted_iota(jnp.int32, (W, sW), 0).astype(jnp.float32)
    wout = lax.broadcasted_iota(jnp.int32, (W, sW), 1).astype(jnp.float32)
    src_w = jnp.minimum(wout * r_w, W - 1)
    mwt = jnp.maximum(0.0, 1.0 - jnp.abs(src_w - win))

    r_h = (H - 1) / (2 * H - 1)
    hi = lax.broadcasted_iota(jnp.int32, (H, H), 0).astype(jnp.float32)
    hk = lax.broadcasted_iota(jnp.int32, (H, H), 1).astype(jnp.float32)
    src_e = jnp.minimum((2.0 * hi) * r_h, H - 1)
    a_e = jnp.maximum(0.0, 1.0 - jnp.abs(src_e - hk))
    src_o = jnp.minimum((2.0 * hi + 1.0) * r_h, H - 1)
    a_o = jnp.maximum(0.0, 1.0 - jnp.abs(src_o - hk))

    bias = b_ref[0]
    for g in range(G):
        acc = x_ref[g, 0].astype(jnp.float32) * w_ref[0] + bias
        for c in range(1, C):
            acc = acc + x_ref[g, c].astype(jnp.float32) * w_ref[c]

        mid = jnp.dot(acc, mwt, preferred_element_type=jnp.float32)
        e_rows = jnp.dot(a_e, mid, preferred_element_type=jnp.float32)
        o_rows = jnp.dot(a_o, mid, preferred_element_type=jnp.float32)
        o_ref[g, 0, :, 0:sW] = e_rows.astype(o_ref.dtype)
        o_ref[g, 0, :, sW:2 * sW] = o_rows.astype(o_ref.dtype)


def kernel(w, b, x):
    B, C, H, W = x.shape
    sH, sW = 2 * H, 2 * W
    G = _IMGS_PER_STEP if B % _IMGS_PER_STEP == 0 else 1
    wv = w.astype(jnp.float32).reshape(C)
    bv = b.astype(jnp.float32).reshape(1)
    smem = pl.BlockSpec(memory_space=pltpu.MemorySpace.SMEM)
    out = pl.pallas_call(
        functools.partial(_ph_kernel, G, C, H, W, sW),
        out_shape=jax.ShapeDtypeStruct((B, 1, H, 2 * sW), jnp.float32),
        grid=(B // G,),
        in_specs=[pl.BlockSpec((G, C, H, W), lambda i: (i, 0, 0, 0)),
                  smem, smem],
        out_specs=pl.BlockSpec((G, 1, H, 2 * sW), lambda i: (i, 0, 0, 0)),
        compiler_params=pltpu.CompilerParams(
            dimension_semantics=("parallel",),
            vmem_limit_bytes=96 * 1024 * 1024),
    )(x, wv, bv)
    return out.reshape(B, 1, sH, sW)


# speedup vs baseline: 4.2417x; 2.4711x over previous
import functools

import jax
import jax.numpy as jnp
from jax import lax
from jax.experimental import pallas as pl
from jax.experimental.pallas import tpu as pltpu


_IMGS_PER_STEP = 8


def _ph_kernel(G, C, H, W, sW, x_ref, w_ref, b_ref, o_ref):
    r_w = (W - 1) / (sW - 1)
    win = lax.broadcasted_iota(jnp.int32, (W, sW), 0).astype(jnp.float32)
    wout = lax.broadcasted_iota(jnp.int32, (W, sW), 1).astype(jnp.float32)
    src_w = jnp.minimum(wout * r_w, W - 1)
    mwt = jnp.maximum(0.0, 1.0 - jnp.abs(src_w - win))

    sH = 2 * H
    r_h = (H - 1) / (sH - 1)
    hi = lax.broadcasted_iota(jnp.int32, (sH, H), 0).astype(jnp.float32)
    hk = lax.broadcasted_iota(jnp.int32, (sH, H), 1).astype(jnp.float32)
    src_h = jnp.minimum(hi * r_h, H - 1)
    a_h = jnp.maximum(0.0, 1.0 - jnp.abs(src_h - hk))

    bias = b_ref[0]
    for g in range(G):
        acc = x_ref[g, 0].astype(jnp.float32) * w_ref[0] + bias
        for c in range(1, C):
            acc = acc + x_ref[g, c].astype(jnp.float32) * w_ref[c]

        mid = jnp.dot(acc, mwt, preferred_element_type=jnp.float32)
        o_ref[g, 0] = jnp.dot(a_h, mid,
                              preferred_element_type=jnp.float32
                              ).astype(o_ref.dtype)


def kernel(w, b, x):
    B, C, H, W = x.shape
    sH, sW = 2 * H, 2 * W
    G = _IMGS_PER_STEP if B % _IMGS_PER_STEP == 0 else 1
    wv = w.astype(jnp.float32).reshape(C)
    bv = b.astype(jnp.float32).reshape(1)
    smem = pl.BlockSpec(memory_space=pltpu.MemorySpace.SMEM)
    return pl.pallas_call(
        functools.partial(_ph_kernel, G, C, H, W, sW),
        out_shape=jax.ShapeDtypeStruct((B, 1, sH, sW), jnp.float32),
        grid=(B // G,),
        in_specs=[pl.BlockSpec((G, C, H, W), lambda i: (i, 0, 0, 0)),
                  smem, smem],
        out_specs=pl.BlockSpec((G, 1, sH, sW), lambda i: (i, 0, 0, 0)),
        compiler_params=pltpu.CompilerParams(
            dimension_semantics=("parallel",),
            vmem_limit_bytes=96 * 1024 * 1024),
    )(x, wv, bv)
